# baseline (speedup 1.0000x reference)
import functools

import jax
import jax.numpy as jnp
from jax import lax
from jax.experimental import pallas as pl
from jax.experimental.pallas import tpu as pltpu
from jax.experimental.pallas import tpu_sc as plsc

B = 4096
L = 200
F = 64
EPS = 1e-5

NC = 2
NS = 16
NW = NC * NS
ROWS_PER_W = B // NW
LCH = 100
NCH = L // LCH
FV = F // 16


def _sc_pool_body(x_hbm, table_hbm, out_hbm, idx_v, rows_v, pooled_v, sem):
    wid = lax.axis_index("s") * NC + lax.axis_index("c")
    base = wid * ROWS_PER_W
    inv_l = 1.0 / L

    def row_body(i, _):
        pltpu.sync_copy(x_hbm.at[base + i], idx_v)
        accs = [jnp.zeros((16,), jnp.float32) for _ in range(FV)]
        for ch in range(NCH):
            pltpu.async_copy(table_hbm.at[idx_v.at[ch]], rows_v, sem).wait()

            def acc_body(j, accs):
                return tuple(
                    accs[c] + rows_v[j, pl.ds(16 * c, 16)] for c in range(FV)
                )

            accs = list(lax.fori_loop(0, LCH, acc_body, tuple(accs)))
        for c in range(FV):
            pooled_v[i, pl.ds(16 * c, 16)] = accs[c] * inv_l
        return 0

    lax.fori_loop(0, ROWS_PER_W, row_body, 0)
    pltpu.sync_copy(pooled_v, out_hbm.at[pl.ds(base, ROWS_PER_W)])


_sc_pool = functools.partial(
    pl.kernel,
    out_type=jax.ShapeDtypeStruct((B, F), jnp.float32),
    mesh=plsc.VectorSubcoreMesh(core_axis_name="c", subcore_axis_name="s"),
    scratch_types=[
        pltpu.VMEM((NCH, LCH), jnp.int32),
        pltpu.VMEM((LCH, F), jnp.float32),
        pltpu.VMEM((ROWS_PER_W, F), jnp.float32),
        pltpu.SemaphoreType.DMA,
    ],
    compiler_params=pltpu.CompilerParams(use_tc_tiling_on_sc=False),
)(_sc_pool_body)


def _tc_head_body(p_ref, w_ref, b_ref, bng_ref, bnb_ref, lng_ref, lnb_ref,
                  out_ref):
    p = p_ref[...]
    h = lax.dot_general(p, w_ref[...], (((1,), (1,)), ((), ())),
                        preferred_element_type=jnp.float32) + b_ref[...]
    mu = jnp.mean(h, axis=0, keepdims=True)
    var = jnp.mean((h - mu) ** 2, axis=0, keepdims=True)
    h = (h - mu) * lax.rsqrt(var + EPS) * bng_ref[...] + bnb_ref[...]
    lmu = jnp.mean(h, axis=1, keepdims=True)
    lvar = jnp.mean((h - lmu) ** 2, axis=1, keepdims=True)
    out_ref[...] = ((h - lmu) * lax.rsqrt(lvar + EPS) * lng_ref[...]
                    + lnb_ref[...])


def _tc_head(pooled, W, b, bn_gamma, bn_beta, ln_gamma, ln_beta):
    return pl.pallas_call(
        _tc_head_body,
        out_shape=jax.ShapeDtypeStruct((B, F), jnp.float32),
    )(pooled, W, b.reshape(1, F), bn_gamma.reshape(1, F),
      bn_beta.reshape(1, F), ln_gamma.reshape(1, F), ln_beta.reshape(1, F))


def kernel(x, table, W, b, bn_gamma, bn_beta, ln_gamma, ln_beta):
    x3 = x.astype(jnp.int32).reshape(B, NCH, LCH)
    pooled = _sc_pool(x3, table)
    return _tc_head(pooled, W, b, bn_gamma, bn_beta, ln_gamma, ln_beta)


# speedup vs baseline: 1.3864x; 1.3864x over previous
import functools

import jax
import jax.numpy as jnp
from jax import lax
from jax.experimental import pallas as pl
from jax.experimental.pallas import tpu as pltpu
from jax.experimental.pallas import tpu_sc as plsc

B = 4096
L = 200
F = 64
EPS = 1e-5

NC = 2
NS = 16
NW = NC * NS
ROWS_PER_W = B // NW
LCH = 100
NCH = L // LCH
K = ROWS_PER_W * NCH
NBUF = 4
FV = F // 16


def _acc_chunk(buf, accs):
    def body(j, accs):
        return tuple(accs[c] + buf[j, pl.ds(16 * c, 16)] for c in range(FV))
    return lax.fori_loop(0, LCH, body, accs, unroll=2)


def _sc_pool_body(x_hbm, table_hbm, out_hbm, idx_all, b0, b1, b2, b3,
                  pooled_v, s0, s1, s2, s3):
    bufs = (b0, b1, b2, b3)
    sems = (s0, s1, s2, s3)
    wid = lax.axis_index("s") * NC + lax.axis_index("c")
    inv_l = 1.0 / L

    pltpu.sync_copy(x_hbm.at[wid], idx_all)

    handles = [
        pltpu.async_copy(table_hbm.at[idx_all.at[b]], bufs[b], sems[b])
        for b in range(NBUF)
    ]
    accs = tuple(jnp.zeros((16,), jnp.float32) for _ in range(FV))
    for k in range(K):
        b = k % NBUF
        handles[b].wait()
        accs = _acc_chunk(bufs[b], accs)
        if k + NBUF < K:
            handles[b] = pltpu.async_copy(
                table_hbm.at[idx_all.at[k + NBUF]], bufs[b], sems[b])
        if k % NCH == NCH - 1:
            row = k // NCH
            for c in range(FV):
                pooled_v[row, pl.ds(16 * c, 16)] = accs[c] * inv_l
            accs = tuple(jnp.zeros((16,), jnp.float32) for _ in range(FV))

    pltpu.sync_copy(pooled_v, out_hbm.at[pl.ds(wid * ROWS_PER_W, ROWS_PER_W)])


_sc_pool = functools.partial(
    pl.kernel,
    out_type=jax.ShapeDtypeStruct((B, F), jnp.float32),
    mesh=plsc.VectorSubcoreMesh(core_axis_name="c", subcore_axis_name="s"),
    scratch_types=[
        pltpu.VMEM((K, LCH), jnp.int32),
        pltpu.VMEM((LCH, F), jnp.float32),
        pltpu.VMEM((LCH, F), jnp.float32),
        pltpu.VMEM((LCH, F), jnp.float32),
        pltpu.VMEM((LCH, F), jnp.float32),
        pltpu.VMEM((ROWS_PER_W, F), jnp.float32),
        pltpu.SemaphoreType.DMA,
        pltpu.SemaphoreType.DMA,
        pltpu.SemaphoreType.DMA,
        pltpu.SemaphoreType.DMA,
    ],
    compiler_params=pltpu.CompilerParams(use_tc_tiling_on_sc=False),
)(_sc_pool_body)


def _tc_head_body(p_ref, w_ref, b_ref, bng_ref, bnb_ref, lng_ref, lnb_ref,
                  out_ref):
    p = p_ref[...]
    h = lax.dot_general(p, w_ref[...], (((1,), (1,)), ((), ())),
                        preferred_element_type=jnp.float32) + b_ref[...]
    mu = jnp.mean(h, axis=0, keepdims=True)
    var = jnp.mean((h - mu) ** 2, axis=0, keepdims=True)
    h = (h - mu) * lax.rsqrt(var + EPS) * bng_ref[...] + bnb_ref[...]
    lmu = jnp.mean(h, axis=1, keepdims=True)
    lvar = jnp.mean((h - lmu) ** 2, axis=1, keepdims=True)
    out_ref[...] = ((h - lmu) * lax.rsqrt(lvar + EPS) * lng_ref[...]
                    + lnb_ref[...])


def _tc_head(pooled, W, b, bn_gamma, bn_beta, ln_gamma, ln_beta):
    return pl.pallas_call(
        _tc_head_body,
        out_shape=jax.ShapeDtypeStruct((B, F), jnp.float32),
    )(pooled, W, b.reshape(1, F), bn_gamma.reshape(1, F),
      bn_beta.reshape(1, F), ln_gamma.reshape(1, F), ln_beta.reshape(1, F))


def kernel(x, table, W, b, bn_gamma, bn_beta, ln_gamma, ln_beta):
    x3 = x.astype(jnp.int32).reshape(NW, K, LCH)
    pooled = _sc_pool(x3, table)
    return _tc_head(pooled, W, b, bn_gamma, bn_beta, ln_gamma, ln_beta)
